# baseline (speedup 1.0000x reference)
import jax
import jax.numpy as jnp
from jax.experimental import pallas as pl
from jax.experimental.pallas import tpu as pltpu

_BLOCK_ROWS = 512


def _router_body(x_ref, w_ref, b_ref, out_ref):
    x = x_ref[...]
    w = w_ref[...]
    logits = jax.lax.dot_general(
        x, w, (((1,), (0,)), ((), ())), preferred_element_type=jnp.float32
    ) + b_ref[...][None, :]
    m = jnp.max(logits, axis=-1, keepdims=True)
    e = jnp.exp(logits - m)
    out_ref[...] = e / jnp.sum(e, axis=-1, keepdims=True)


def kernel(x_t, mem, W, b):
    n, d = x_t.shape
    n_exp = W.shape[1]
    grid = (n // _BLOCK_ROWS,)
    probs = pl.pallas_call(
        _router_body,
        grid=grid,
        in_specs=[
            pl.BlockSpec((_BLOCK_ROWS, d), lambda i: (i, 0)),
            pl.BlockSpec((d, n_exp), lambda i: (0, 0)),
            pl.BlockSpec((n_exp,), lambda i: (0,)),
        ],
        out_specs=pl.BlockSpec((_BLOCK_ROWS, n_exp), lambda i: (i, 0)),
        out_shape=jax.ShapeDtypeStruct((n, n_exp), jnp.float32),
    )(x_t, W, b)
    return (probs, mem)


# speedup vs baseline: 1.3084x; 1.3084x over previous
import jax
import jax.numpy as jnp
from jax.experimental import pallas as pl
from jax.experimental.pallas import tpu as pltpu

_BLOCK_ROWS = 4096


def _router_body(x_ref, w_ref, b_ref, out_ref):
    x = x_ref[...]
    w = w_ref[...]
    logits = jax.lax.dot_general(
        x, w, (((1,), (0,)), ((), ())), preferred_element_type=jnp.float32
    ) + b_ref[...][None, :]
    m = jnp.max(logits, axis=-1, keepdims=True)
    e = jnp.exp(logits - m)
    out_ref[...] = e / jnp.sum(e, axis=-1, keepdims=True)


def kernel(x_t, mem, W, b):
    n, d = x_t.shape
    n_exp = W.shape[1]
    grid = (n // _BLOCK_ROWS,)
    probs = pl.pallas_call(
        _router_body,
        grid=grid,
        in_specs=[
            pl.BlockSpec((_BLOCK_ROWS, d), lambda i: (i, 0)),
            pl.BlockSpec((d, n_exp), lambda i: (0, 0)),
            pl.BlockSpec((n_exp,), lambda i: (0,)),
        ],
        out_specs=pl.BlockSpec((_BLOCK_ROWS, n_exp), lambda i: (i, 0)),
        out_shape=jax.ShapeDtypeStruct((n, n_exp), jnp.float32),
    )(x_t, W, b)
    return (probs, mem)
